# initial kernel scaffold (unmeasured)
import jax
import jax.numpy as jnp
from jax import lax
from jax.experimental import pallas as pl
from jax.experimental.pallas import tpu as pltpu


def kernel(
    x,
):
    def body(*refs):
        pass

    out_shape = jax.ShapeDtypeStruct(..., jnp.float32)
    return pl.pallas_call(body, out_shape=out_shape)(...)


# baseline (device time: 26012 ns/iter reference)
import jax
import jax.numpy as jnp
from jax import lax
from jax.experimental import pallas as pl
from jax.experimental.pallas import tpu as pltpu

P = 4


def kernel(x):
    m, n = x.shape
    blk = n // P
    out_rows = P * m

    def body(x_ref, out_ref, comm_ref, send_sems, recv_sems):
        my_x = lax.axis_index("x")
        my_y = lax.axis_index("y")
        my_z = lax.axis_index("z")

        barrier_sem = pltpu.get_barrier_semaphore()
        for d in range(1, P):
            pl.semaphore_signal(
                barrier_sem,
                inc=1,
                device_id=(my_x, my_y, (my_z + d) % P),
                device_id_type=pl.DeviceIdType.MESH,
            )
        pl.semaphore_wait(barrier_sem, P - 1)

        comm_ref[:, :] = x_ref[:, :].astype(comm_ref.dtype)

        rdmas = []
        for d in range(1, P):
            dst_z = (my_z + d) % P
            rdma = pltpu.make_async_remote_copy(
                src_ref=comm_ref.at[:, pl.ds(dst_z * blk, blk)],
                dst_ref=out_ref.at[pl.ds(my_z * m, m), :],
                send_sem=send_sems.at[d - 1],
                recv_sem=recv_sems.at[d - 1],
                device_id=(my_x, my_y, dst_z),
                device_id_type=pl.DeviceIdType.MESH,
            )
            rdma.start()
            rdmas.append(rdma)

        out_ref[pl.ds(my_z * m, m), :] = comm_ref[:, pl.ds(my_z * blk, blk)]

        for rdma in rdmas:
            rdma.wait()

    return pl.pallas_call(
        body,
        out_shape=jax.ShapeDtypeStruct((out_rows, blk), jnp.bfloat16),
        in_specs=[pl.BlockSpec(memory_space=pltpu.VMEM)],
        out_specs=pl.BlockSpec(memory_space=pltpu.VMEM),
        scratch_shapes=[
            pltpu.VMEM((m, n), jnp.bfloat16),
            pltpu.SemaphoreType.DMA((P - 1,)),
            pltpu.SemaphoreType.DMA((P - 1,)),
        ],
        compiler_params=pltpu.CompilerParams(collective_id=0),
    )(x)


# device time: 25917 ns/iter; 1.0037x vs baseline; 1.0037x over previous
import jax
import jax.numpy as jnp
from jax import lax
from jax.experimental import pallas as pl
from jax.experimental.pallas import tpu as pltpu

P = 4


def kernel(x):
    m, n = x.shape
    blk = n // P
    out_rows = P * m

    SEND_ORDER = (2, 3, 1)

    def body(x_ref, out_ref, comm_ref, send_sems, recv_sems):
        my_x = lax.axis_index("x")
        my_y = lax.axis_index("y")
        my_z = lax.axis_index("z")

        barrier_sem = pltpu.get_barrier_semaphore()
        for d in range(1, P):
            pl.semaphore_signal(
                barrier_sem,
                inc=1,
                device_id=(my_x, my_y, (my_z + d) % P),
                device_id_type=pl.DeviceIdType.MESH,
            )

        for i, d in enumerate(SEND_ORDER):
            dst_z = (my_z + d) % P
            comm_ref[i, :, :] = x_ref[:, pl.ds(dst_z * blk, blk)].astype(
                comm_ref.dtype
            )

        pl.semaphore_wait(barrier_sem, P - 1)

        rdmas = []
        for i, d in enumerate(SEND_ORDER):
            dst_z = (my_z + d) % P
            rdma = pltpu.make_async_remote_copy(
                src_ref=comm_ref.at[i],
                dst_ref=out_ref.at[pl.ds(my_z * m, m), :],
                send_sem=send_sems.at[d - 1],
                recv_sem=recv_sems.at[d - 1],
                device_id=(my_x, my_y, dst_z),
                device_id_type=pl.DeviceIdType.MESH,
            )
            rdma.start()
            rdmas.append(rdma)

        out_ref[pl.ds(my_z * m, m), :] = x_ref[
            :, pl.ds(my_z * blk, blk)
        ].astype(out_ref.dtype)

        for rdma in rdmas:
            rdma.wait()

    return pl.pallas_call(
        body,
        out_shape=jax.ShapeDtypeStruct((out_rows, blk), jnp.bfloat16),
        in_specs=[pl.BlockSpec(memory_space=pltpu.VMEM)],
        out_specs=pl.BlockSpec(memory_space=pltpu.VMEM),
        scratch_shapes=[
            pltpu.VMEM((P - 1, m, blk), jnp.bfloat16),
            pltpu.SemaphoreType.DMA((P - 1,)),
            pltpu.SemaphoreType.DMA((P - 1,)),
        ],
        compiler_params=pltpu.CompilerParams(collective_id=0),
    )(x)


# device time: 25799 ns/iter; 1.0083x vs baseline; 1.0046x over previous
import jax
import jax.numpy as jnp
from jax import lax
from jax.experimental import pallas as pl
from jax.experimental.pallas import tpu as pltpu

P = 4


def kernel(x):
    m, n = x.shape
    blk = n // P
    out_rows = P * m

    def body(x_ref, out_ref, comm_ref, send_sems, recv_sems):
        my_x = lax.axis_index("x")
        my_y = lax.axis_index("y")
        my_z = lax.axis_index("z")

        barrier_sem = pltpu.get_barrier_semaphore()
        for d in range(1, P):
            pl.semaphore_signal(
                barrier_sem,
                inc=1,
                device_id=(my_x, my_y, (my_z + d) % P),
                device_id_type=pl.DeviceIdType.MESH,
            )

        comm_ref[:, :] = x_ref[:, :].astype(comm_ref.dtype)

        pl.semaphore_wait(barrier_sem, P - 1)

        rdmas = []
        for d in range(1, P):
            dst_z = (my_z + d) % P
            rdma = pltpu.make_async_remote_copy(
                src_ref=comm_ref.at[:, pl.ds(dst_z * blk, blk)],
                dst_ref=out_ref.at[pl.ds(my_z * m, m), :],
                send_sem=send_sems.at[d - 1],
                recv_sem=recv_sems.at[d - 1],
                device_id=(my_x, my_y, dst_z),
                device_id_type=pl.DeviceIdType.MESH,
            )
            rdma.start()
            rdmas.append(rdma)

        out_ref[pl.ds(my_z * m, m), :] = comm_ref[:, pl.ds(my_z * blk, blk)]

        for rdma in rdmas:
            rdma.wait()

    return pl.pallas_call(
        body,
        out_shape=jax.ShapeDtypeStruct((out_rows, blk), jnp.bfloat16),
        in_specs=[pl.BlockSpec(memory_space=pltpu.VMEM)],
        out_specs=pl.BlockSpec(memory_space=pltpu.VMEM),
        scratch_shapes=[
            pltpu.VMEM((m, n), jnp.bfloat16),
            pltpu.SemaphoreType.DMA((P - 1,)),
            pltpu.SemaphoreType.DMA((P - 1,)),
        ],
        compiler_params=pltpu.CompilerParams(collective_id=0),
    )(x)
